# baseline (speedup 1.0000x reference)
import functools

import jax
import jax.numpy as jnp
from jax import lax
from jax.experimental import pallas as pl
from jax.experimental.pallas import tpu as pltpu
from jax.experimental.pallas import tpu_sc as plsc

NC, NS, L = 2, 16, 16
NW = NC * NS
BATCH = 16384
DIM = 64
BPW = BATCH // NW
CH = 128
NCH = BPW // CH
NG = CH // L


def _fire(ent_p, rel_p, pidx, bufs, ring, ch, sems):
    srcs = (ent_p, rel_p, ent_p)
    return [
        pltpu.async_copy(srcs[t].at[pidx[t].at[ch]], bufs[ring * 3 + t],
                         sems[ring])
        for t in range(3)
    ]


def _tec_body(heads, rels, tails, ent_p, rel_p, out, *scratch):
    ids = scratch[0:3]
    pidx = scratch[3:6]
    bufs = scratch[6:12]
    outv = scratch[12]
    sems = scratch[13:15]
    sem_i = scratch[15]

    wid = lax.axis_index("s") * NC + lax.axis_index("c")
    base = wid * BPW

    srcs_hbm = (heads, rels, tails)
    cps = []
    for ch in range(NCH):
        for t in range(3):
            cps.append(pltpu.async_copy(
                srcs_hbm[t].at[pl.ds(base + ch * CH, CH)], ids[t].at[ch], sem_i))
    for cp in cps:
        cp.wait()

    one = jnp.int32(1)
    for ch in range(NCH):
        for t in range(3):
            for v in range(CH // L):
                sl = pl.ds(v * L, L)
                pidx[t][ch, sl] = lax.shift_right_logical(ids[t][ch, sl], one)

    lane = lax.iota(jnp.int32, L)

    pending = {0: _fire(ent_p, rel_p, pidx, bufs, 0, 0, sems)}

    for ch in range(NCH):
        ring = ch % 2
        for cp in pending.pop(ring):
            cp.wait()
        if ch + 1 < NCH:
            pending[1 - ring] = _fire(ent_p, rel_p, pidx, bufs, 1 - ring,
                                      ch + 1, sems)

        hbuf, rbuf, tbuf = bufs[ring * 3], bufs[ring * 3 + 1], bufs[ring * 3 + 2]

        rows = [lane + jnp.int32(g * L) for g in range(NG)]
        halves = []
        for g in range(NG):
            sl = pl.ds(g * L, L)
            halves.append([
                lax.shift_left((ids[t][ch, sl] & one), jnp.int32(6))
                for t in range(3)
            ])

        def cstep(c, accs):
            rot = (lane + c) & jnp.int32(DIM - 1)
            out_accs = []
            for g in range(NG):
                h = plsc.load_gather(hbuf, [rows[g], halves[g][0] + rot])
                r = plsc.load_gather(rbuf, [rows[g], halves[g][1] + rot])
                t_ = plsc.load_gather(tbuf, [rows[g], halves[g][2] + rot])
                d = h + r - t_
                out_accs.append(accs[g] + d * d)
            return tuple(out_accs)

        zero = jnp.zeros((L,), jnp.float32)
        accs = lax.fori_loop(0, DIM, cstep, (zero,) * NG)

        for g in range(NG):
            x = accs[g]
            seed = jnp.int32(0x5F3759DF) - (plsc.bitcast(x, jnp.int32) >> 1)
            y = plsc.bitcast(seed, jnp.float32)
            for _ in range(3):
                y = y * (jnp.float32(1.5) - jnp.float32(0.5) * x * y * y)
            outv[pl.ds(ch * CH + g * L, L)] = -(x * y)

    pltpu.sync_copy(outv, out.at[pl.ds(base, BPW)])


_transe = functools.partial(
    pl.kernel,
    out_type=jax.ShapeDtypeStruct((BATCH,), jnp.float32),
    mesh=plsc.VectorSubcoreMesh(core_axis_name="c", subcore_axis_name="s"),
    compiler_params=pltpu.CompilerParams(
        needs_layout_passes=False, use_tc_tiling_on_sc=True),
    scratch_types=(
        [pltpu.VMEM((NCH, CH), jnp.int32)] * 3
        + [pltpu.VMEM((NCH, CH), jnp.int32)] * 3
        + [pltpu.VMEM((CH, 128), jnp.float32)] * 6
        + [pltpu.VMEM((BPW,), jnp.float32)]
        + [pltpu.SemaphoreType.DMA] * 2
        + [pltpu.SemaphoreType.DMA]
    ),
)(_tec_body)


@jax.jit
def kernel(heads, rels, tails, entity_emb, relation_emb):
    ent_p = jnp.reshape(entity_emb, (entity_emb.shape[0] // 2, 2 * DIM))
    rel_p = jnp.reshape(relation_emb, (relation_emb.shape[0] // 2, 2 * DIM))
    return _transe(heads, rels, tails, ent_p, rel_p)


# speedup vs baseline: 1.7051x; 1.7051x over previous
import functools

import jax
import jax.numpy as jnp
from jax import lax
from jax.experimental import pallas as pl
from jax.experimental.pallas import tpu as pltpu
from jax.experimental.pallas import tpu_sc as plsc

NC, NS, L = 2, 16, 16
NW = NC * NS
BATCH = 16384
DIM = 64
BPW = BATCH // NW
CH = 128
NCH = BPW // CH
NG = CH // L


def _fire(ent, rel, hS, rS, tS, bufs, ring, ch, sems):
    hbuf, rbuf, tbuf = bufs[ring * 3:ring * 3 + 3]
    sem = sems[ring]

    def body(j, carry):
        ids16 = [ref[pl.ds(ch * CH + j * L, L)] for ref in (hS, rS, tS)]
        for k in range(L):
            i = j * L + k
            pltpu.async_copy(ent.at[ids16[0][k]], hbuf.at[i], sem)
            pltpu.async_copy(rel.at[ids16[1][k]], rbuf.at[i], sem)
            pltpu.async_copy(ent.at[ids16[2][k]], tbuf.at[i], sem)
        return carry

    lax.fori_loop(0, CH // L, body, 0)


def _drain(ent, bufs, ring, sems):
    for buf in bufs[ring * 3:ring * 3 + 3]:
        pltpu.make_async_copy(ent.at[pl.ds(0, CH)], buf, sems[ring]).wait()


def _tec_body(heads, rels, tails, ent, rel, out, *scratch):
    hS, rS, tS = scratch[0:3]
    bufs = scratch[3:9]
    outv = scratch[9]
    sems = scratch[10:12]
    sem_i = scratch[12]

    wid = lax.axis_index("s") * NC + lax.axis_index("c")
    base = wid * BPW

    cps = [
        pltpu.async_copy(heads.at[pl.ds(base, BPW)], hS, sem_i),
        pltpu.async_copy(rels.at[pl.ds(base, BPW)], rS, sem_i),
        pltpu.async_copy(tails.at[pl.ds(base, BPW)], tS, sem_i),
    ]
    for cp in cps:
        cp.wait()

    lane = lax.iota(jnp.int32, L)
    rows = [lane + jnp.int32(g * L) for g in range(NG)]

    _fire(ent, rel, hS, rS, tS, bufs, 0, 0, sems)

    for ch in range(NCH):
        ring = ch % 2
        _drain(ent, bufs, ring, sems)
        if ch + 1 < NCH:
            _fire(ent, rel, hS, rS, tS, bufs, 1 - ring, ch + 1, sems)

        hbuf, rbuf, tbuf = bufs[ring * 3:ring * 3 + 3]

        def cstep(c, accs):
            rot = (lane + c) & jnp.int32(DIM - 1)
            out_accs = []
            for g in range(NG):
                h = plsc.load_gather(hbuf, [rows[g], rot])
                r = plsc.load_gather(rbuf, [rows[g], rot])
                t_ = plsc.load_gather(tbuf, [rows[g], rot])
                d = h + r - t_
                out_accs.append(accs[g] + d * d)
            return tuple(out_accs)

        zero = jnp.zeros((L,), jnp.float32)
        accs = lax.fori_loop(0, DIM, cstep, (zero,) * NG)

        for g in range(NG):
            x = accs[g]
            seed = jnp.int32(0x5F3759DF) - (plsc.bitcast(x, jnp.int32) >> 1)
            y = plsc.bitcast(seed, jnp.float32)
            for _ in range(3):
                y = y * (jnp.float32(1.5) - jnp.float32(0.5) * x * y * y)
            outv[pl.ds(ch * CH + g * L, L)] = -(x * y)

    pltpu.sync_copy(outv, out.at[pl.ds(base, BPW)])


_transe = functools.partial(
    pl.kernel,
    out_type=jax.ShapeDtypeStruct((BATCH,), jnp.float32),
    mesh=plsc.VectorSubcoreMesh(core_axis_name="c", subcore_axis_name="s"),
    compiler_params=pltpu.CompilerParams(
        needs_layout_passes=False, use_tc_tiling_on_sc=True),
    scratch_types=(
        [pltpu.VMEM((BPW,), jnp.int32)] * 3
        + [pltpu.VMEM((CH, DIM), jnp.float32)] * 6
        + [pltpu.VMEM((BPW,), jnp.float32)]
        + [pltpu.SemaphoreType.DMA] * 2
        + [pltpu.SemaphoreType.DMA]
    ),
)(_tec_body)


@jax.jit
def kernel(heads, rels, tails, entity_emb, relation_emb):
    return _transe(heads, rels, tails, entity_emb, relation_emb)


# speedup vs baseline: 2.5296x; 1.4835x over previous
import functools

import jax
import jax.numpy as jnp
from jax import lax
from jax.experimental import pallas as pl
from jax.experimental.pallas import tpu as pltpu
from jax.experimental.pallas import tpu_sc as plsc

NC, NS, L = 2, 16, 16
NW = NC * NS
BATCH = 16384
DIM = 64
BPW = BATCH // NW
CH = 128
NCH = BPW // CH
NG = CH // L
CQ = CH // 8


def _fire(ent3, rel3, hS, rS, tS, bufs, ring, ch, sems):
    hbuf, rbuf, tbuf = bufs[ring * 3:ring * 3 + 3]
    sem = sems[ring]
    three = jnp.int32(3)
    seven = jnp.int32(7)

    def body(j, carry):
        ids16 = [ref[pl.ds(ch * CH + j * L, L)] for ref in (hS, rS, tS)]
        j2 = j * 2
        for k in range(L):
            q = j2 + (k >> 3)
            s = k & 7
            idh, idr, idt = ids16[0][k], ids16[1][k], ids16[2][k]
            pltpu.async_copy(ent3.at[idh >> three, idh & seven],
                             hbuf.at[q, s], sem)
            pltpu.async_copy(rel3.at[idr >> three, idr & seven],
                             rbuf.at[q, s], sem)
            pltpu.async_copy(ent3.at[idt >> three, idt & seven],
                             tbuf.at[q, s], sem)
        return carry

    lax.fori_loop(0, CH // L, body, 0)


def _drain(ent3, bufs, ring, sems):
    for buf in bufs[ring * 3:ring * 3 + 3]:
        pltpu.make_async_copy(ent3.at[pl.ds(0, CQ)], buf, sems[ring]).wait()


def _tec_body(heads, rels, tails, ent3, rel3, out, *scratch):
    hS, rS, tS = scratch[0:3]
    bufs = scratch[3:9]
    outv = scratch[9]
    sems = scratch[10:12]
    sem_i = scratch[12]

    wid = lax.axis_index("s") * NC + lax.axis_index("c")
    base = wid * BPW

    cps = [
        pltpu.async_copy(heads.at[pl.ds(base, BPW)], hS, sem_i),
        pltpu.async_copy(rels.at[pl.ds(base, BPW)], rS, sem_i),
        pltpu.async_copy(tails.at[pl.ds(base, BPW)], tS, sem_i),
    ]
    for cp in cps:
        cp.wait()

    lane = lax.iota(jnp.int32, L)
    qv = [(lane + jnp.int32(g * L)) >> jnp.int32(3) for g in range(NG)]
    sv = lane & jnp.int32(7)

    _fire(ent3, rel3, hS, rS, tS, bufs, 0, 0, sems)

    for ch in range(NCH):
        ring = ch % 2
        _drain(ent3, bufs, ring, sems)
        if ch + 1 < NCH:
            _fire(ent3, rel3, hS, rS, tS, bufs, 1 - ring, ch + 1, sems)

        hbuf, rbuf, tbuf = bufs[ring * 3:ring * 3 + 3]

        def cstep(c, accs):
            rot = (lane + c) & jnp.int32(DIM - 1)
            out_accs = []
            for g in range(NG):
                h = plsc.load_gather(hbuf, [qv[g], sv, rot])
                r = plsc.load_gather(rbuf, [qv[g], sv, rot])
                t_ = plsc.load_gather(tbuf, [qv[g], sv, rot])
                d = h + r - t_
                out_accs.append(accs[g] + d * d)
            return tuple(out_accs)

        zero = jnp.zeros((L,), jnp.float32)
        accs = lax.fori_loop(0, DIM, cstep, (zero,) * NG)

        for g in range(NG):
            x = accs[g]
            seed = jnp.int32(0x5F3759DF) - (plsc.bitcast(x, jnp.int32) >> 1)
            y = plsc.bitcast(seed, jnp.float32)
            for _ in range(3):
                y = y * (jnp.float32(1.5) - jnp.float32(0.5) * x * y * y)
            outv[pl.ds(ch * CH + g * L, L)] = -(x * y)

    pltpu.sync_copy(outv, out.at[pl.ds(base, BPW)])


_transe = functools.partial(
    pl.kernel,
    out_type=jax.ShapeDtypeStruct((BATCH,), jnp.float32),
    mesh=plsc.VectorSubcoreMesh(core_axis_name="c", subcore_axis_name="s"),
    compiler_params=pltpu.CompilerParams(
        needs_layout_passes=False, use_tc_tiling_on_sc=True),
    scratch_types=(
        [pltpu.VMEM((BPW,), jnp.int32)] * 3
        + [pltpu.VMEM((CQ, 8, DIM), jnp.float32)] * 6
        + [pltpu.VMEM((BPW,), jnp.float32)]
        + [pltpu.SemaphoreType.DMA] * 2
        + [pltpu.SemaphoreType.DMA]
    ),
)(_tec_body)


@jax.jit
def kernel(heads, rels, tails, entity_emb, relation_emb):
    ent3 = jnp.reshape(entity_emb, (entity_emb.shape[0] // 8, 8, DIM))
    rel3 = jnp.reshape(relation_emb, (relation_emb.shape[0] // 8, 8, DIM))
    return _transe(heads, rels, tails, ent3, rel3)
